# Initial kernel scaffold; baseline (speedup 1.0000x reference)
import jax
import jax.numpy as jnp
from jax.experimental import pallas as pl


def kernel(x, attn):
    raise NotImplementedError("write your pallas kernel here")


# speedup vs baseline: 1.5071x; 1.5071x over previous
import functools

import jax
import jax.numpy as jnp
import numpy as np
from jax import lax
from jax.experimental import pallas as pl
from jax.experimental.pallas import tpu as pltpu
from jax.experimental.pallas import tpu_sc as plsc

_THRESH = np.float32(0.01)
_RQ = 256
_RX = 256
_L = 16


def _argmax_stage(attn):
    heads, n, _ = attn.shape

    def body(attn_ref, idx_ref, acc_ref):
        h = pl.program_id(1)

        @pl.when(h == 0)
        def _():
            acc_ref[...] = attn_ref[0]

        @pl.when(h > 0)
        def _():
            acc_ref[...] += attn_ref[0]

        @pl.when(h == heads - 1)
        def _():
            mean = acc_ref[...] * jnp.float32(1.0 / heads)
            rowmax = jnp.max(mean, axis=1, keepdims=True)
            iota = lax.broadcasted_iota(jnp.int32, (_RQ, n), 1)
            cand = jnp.where(mean == rowmax, iota, jnp.int32(n))
            idx = jnp.min(cand, axis=1, keepdims=True)
            valid = rowmax >= _THRESH
            idx_ref[...] = jnp.where(valid, idx, jnp.int32(-1))

    return pl.pallas_call(
        body,
        grid=(n // _RQ, heads),
        in_specs=[pl.BlockSpec((1, _RQ, n), lambda q, h: (h, q, 0))],
        out_specs=pl.BlockSpec((_RQ, 1), lambda q, h: (q, 0)),
        out_shape=jax.ShapeDtypeStruct((n, 1), jnp.int32),
        scratch_shapes=[pltpu.VMEM((_RQ, n), jnp.float32)],
    )(attn)


def _make_sc_scatter(n):
    mesh = plsc.VectorSubcoreMesh(core_axis_name="c", subcore_axis_name="s")

    @functools.partial(
        pl.kernel,
        mesh=mesh,
        out_type=jax.ShapeDtypeStruct((n,), jnp.int32),
        scratch_types=[
            pltpu.VMEM((n,), jnp.int32),
            pltpu.VMEM((n,), jnp.int32),
        ],
        compiler_params=pltpu.CompilerParams(needs_layout_passes=False),
    )
    def sc_scatter(idx_hbm, km_hbm, idx_v, km_v):
        cid = lax.axis_index("c")
        sid = lax.axis_index("s")

        @pl.when(jnp.logical_and(cid == 0, sid == 0))
        def _():
            pltpu.sync_copy(idx_hbm, idx_v)

            def zero_body(i, carry):
                km_v[pl.ds(i * _L, _L)] = jnp.zeros((_L,), jnp.int32)
                return carry

            lax.fori_loop(0, n // _L, zero_body, 0)

            def scat_body(i, carry):
                iv = idx_v[pl.ds(i * _L, _L)]
                valid = iv >= 0
                safe = jnp.where(valid, iv, 0)
                plsc.store_scatter(
                    km_v, [safe], jnp.ones((_L,), jnp.int32), mask=valid
                )
                return carry

            lax.fori_loop(0, n // _L, scat_body, 0)
            pltpu.sync_copy(km_v, km_hbm)

    return sc_scatter


def _mask_stage(x, km2d):
    n, m = x.shape

    def body(km_ref, x_ref, o_ref):
        keep = km_ref[...] > 0
        o_ref[...] = jnp.where(keep, x_ref[...], jnp.float32(0.0))

    return pl.pallas_call(
        body,
        grid=(n // _RX,),
        in_specs=[
            pl.BlockSpec((_RX, 1), lambda i: (i, 0)),
            pl.BlockSpec((_RX, m), lambda i: (i, 0)),
        ],
        out_specs=pl.BlockSpec((_RX, m), lambda i: (i, 0)),
        out_shape=jax.ShapeDtypeStruct((n, m), jnp.float32),
    )(km2d, x)


def kernel(x, attn):
    n = x.shape[0]
    idx = _argmax_stage(attn)
    km = _make_sc_scatter(n)(idx.reshape(n))
    return _mask_stage(x, km.reshape(n, 1))


# speedup vs baseline: 2.1530x; 1.4285x over previous
import functools

import jax
import jax.numpy as jnp
import numpy as np
from jax import lax
from jax.experimental import pallas as pl
from jax.experimental.pallas import tpu as pltpu
from jax.experimental.pallas import tpu_sc as plsc

_THRESH = np.float32(0.01)
_RQ = 64
_RX = 256
_L = 16


def _argmax_stage(attn):
    heads, n, _ = attn.shape

    def body(attn_ref, idx_ref):
        b = attn_ref[...]
        acc = b[0]
        for h in range(1, heads):
            acc = acc + b[h]
        mean = acc * jnp.float32(1.0 / heads)
        rowmax = jnp.max(mean, axis=1, keepdims=True)
        iota = lax.broadcasted_iota(jnp.int32, (_RQ, n), 1)
        cand = jnp.where(mean == rowmax, iota, jnp.int32(n))
        idx = jnp.min(cand, axis=1, keepdims=True)
        valid = rowmax >= _THRESH
        idx_ref[...] = jnp.where(valid, idx, jnp.int32(-1))

    return pl.pallas_call(
        body,
        grid=(n // _RQ,),
        in_specs=[pl.BlockSpec((heads, _RQ, n), lambda q: (0, q, 0))],
        out_specs=pl.BlockSpec((_RQ, 1), lambda q: (q, 0)),
        out_shape=jax.ShapeDtypeStruct((n, 1), jnp.int32),
    )(attn)


def _make_sc_scatter(n):
    mesh = plsc.VectorSubcoreMesh(core_axis_name="c", subcore_axis_name="s")

    @functools.partial(
        pl.kernel,
        mesh=mesh,
        out_type=jax.ShapeDtypeStruct((n,), jnp.int32),
        scratch_types=[
            pltpu.VMEM((n,), jnp.int32),
            pltpu.VMEM((n,), jnp.int32),
        ],
        compiler_params=pltpu.CompilerParams(needs_layout_passes=False),
    )
    def sc_scatter(idx_hbm, km_hbm, idx_v, km_v):
        cid = lax.axis_index("c")
        sid = lax.axis_index("s")

        @pl.when(jnp.logical_and(cid == 0, sid == 0))
        def _():
            pltpu.sync_copy(idx_hbm, idx_v)

            def zero_body(i, carry):
                km_v[pl.ds(i * _L, _L)] = jnp.zeros((_L,), jnp.int32)
                return carry

            lax.fori_loop(0, n // _L, zero_body, 0)

            def scat_body(i, carry):
                iv = idx_v[pl.ds(i * _L, _L)]
                valid = iv >= 0
                safe = jnp.where(valid, iv, 0)
                plsc.store_scatter(
                    km_v, [safe], jnp.ones((_L,), jnp.int32), mask=valid
                )
                return carry

            lax.fori_loop(0, n // _L, scat_body, 0)
            pltpu.sync_copy(km_v, km_hbm)

    return sc_scatter


def _mask_stage(x, km2d):
    n, m = x.shape

    def body(km_ref, x_ref, o_ref):
        keep = km_ref[...] > 0
        o_ref[...] = jnp.where(keep, x_ref[...], jnp.float32(0.0))

    return pl.pallas_call(
        body,
        grid=(n // _RX,),
        in_specs=[
            pl.BlockSpec((_RX, 1), lambda i: (i, 0)),
            pl.BlockSpec((_RX, m), lambda i: (i, 0)),
        ],
        out_specs=pl.BlockSpec((_RX, m), lambda i: (i, 0)),
        out_shape=jax.ShapeDtypeStruct((n, m), jnp.float32),
    )(km2d, x)


def kernel(x, attn):
    n = x.shape[0]
    idx = _argmax_stage(attn)
    km = _make_sc_scatter(n)(idx.reshape(n))
    return _mask_stage(x, km.reshape(n, 1))
